# baseline (speedup 1.0000x reference)
import jax
import jax.numpy as jnp
from jax.experimental import pallas as pl
from jax.experimental.pallas import tpu as pltpu

N, D = 16384, 128
BR = 2048


def _rownorm_kernel(s_ref, out_ref):
    out_ref[...] = -0.5 * jnp.sum(s_ref[...], axis=1)


def kernel(Z, labels):
    del labels
    return pl.pallas_call(
        _rownorm_kernel,
        grid=(N // BR,),
        in_specs=[pl.BlockSpec((BR, D), lambda i: (i, 0))],
        out_specs=pl.BlockSpec((BR,), lambda i: (i,)),
        out_shape=jax.ShapeDtypeStruct((N,), Z.dtype),
        compiler_params=pltpu.CompilerParams(
            dimension_semantics=("arbitrary",),
            allow_input_fusion=[True],
        ),
    )(Z * Z)
